# baseline (speedup 1.0000x reference)
import functools
import numpy as np
import jax
import jax.numpy as jnp
from jax import lax
from jax.experimental import pallas as pl
from jax.experimental.pallas import tpu as pltpu
from jax.experimental.pallas import tpu_sc as plsc

N = 10000
E = 320000
NGRAPH = 500
HIDDEN = 128
NFREQ = 10
NLAYERS = 4

NP_ = 10240
EP = 327680
GP = 512
NC = 2
NS = 16
NW = NC * NS
CH = 128
UN = 4
PERW = EP // NW
ROWS_W = PERW // CH
NSUP = ROWS_W // UN


def _silu(v):
    return v * jax.nn.sigmoid(v)


def _sc_gather2(table_a, table_b, idxa2d, idxb2d, width):
    mesh = plsc.VectorSubcoreMesh(core_axis_name="c", subcore_axis_name="s")

    @functools.partial(
        pl.kernel,
        out_type=(
            jax.ShapeDtypeStruct((EP, width), jnp.float32),
            jax.ShapeDtypeStruct((EP, width), jnp.float32),
        ),
        mesh=mesh,
        scratch_types=[
            pltpu.VMEM((ROWS_W, CH), jnp.int32),
            pltpu.VMEM((ROWS_W, CH), jnp.int32),
            pltpu.VMEM((CH, width), jnp.float32),
            pltpu.VMEM((CH, width), jnp.float32),
            pltpu.VMEM((CH, width), jnp.float32),
            pltpu.VMEM((CH, width), jnp.float32),
            pltpu.SemaphoreType.DMA,
            pltpu.SemaphoreType.DMA,
        ],
    )
    def k(ta_h, tb_h, ia_h, ib_h, oa_h, ob_h, ia_v, ib_v,
          ba0, bb0, ba1, bb1, s0, s1):
        cid = lax.axis_index("c")
        sid = lax.axis_index("s")
        wid = sid * NC + cid
        rowbase = wid * ROWS_W
        pltpu.sync_copy(ia_h.at[pl.ds(rowbase, ROWS_W)], ia_v)
        pltpu.sync_copy(ib_h.at[pl.ds(rowbase, ROWS_W)], ib_v)

        def fire(c, ba, bb, sem):
            pltpu.async_copy(ta_h.at[ia_v.at[c]], ba, sem)
            pltpu.async_copy(tb_h.at[ib_v.at[c]], bb, sem)

        def drain(ba, bb, sem):
            pltpu.make_async_copy(ta_h.at[pl.ds(0, CH)], ba, sem).wait()
            pltpu.make_async_copy(ta_h.at[pl.ds(0, CH)], bb, sem).wait()

        fire(0, ba0, bb0, s0)

        def body(j, carry):
            c0 = 2 * j
            c1 = c0 + 1
            fire(c1, ba1, bb1, s1)
            drain(ba0, bb0, s0)
            base0 = (rowbase + c0) * CH
            pltpu.sync_copy(ba0, oa_h.at[pl.ds(base0, CH)])
            pltpu.sync_copy(bb0, ob_h.at[pl.ds(base0, CH)])

            @pl.when(j < ROWS_W // 2 - 1)
            def _():
                fire(c0 + 2, ba0, bb0, s0)

            drain(ba1, bb1, s1)
            base1 = (rowbase + c1) * CH
            pltpu.sync_copy(ba1, oa_h.at[pl.ds(base1, CH)])
            pltpu.sync_copy(bb1, ob_h.at[pl.ds(base1, CH)])
            return carry

        lax.fori_loop(0, ROWS_W // 2, body, 0)

    return k(table_a, table_b, idxa2d, idxb2d)


def _sc_scatter(ef2, idxs2d, zeros128):
    mesh = plsc.VectorSubcoreMesh(core_axis_name="c", subcore_axis_name="s")
    rows_sub = NP_ // NS
    nzc = rows_sub // CH

    @functools.partial(
        pl.kernel,
        out_type=jax.ShapeDtypeStruct((NC, NP_, HIDDEN), jnp.float32),
        mesh=mesh,
        scratch_types=[
            pltpu.VMEM((ROWS_W, CH), jnp.int32),
            pltpu.VMEM((CH, HIDDEN), jnp.float32),
            pltpu.VMEM((CH, HIDDEN), jnp.float32),
            pltpu.VMEM_SHARED((NP_, HIDDEN), jnp.float32),
            pltpu.SemaphoreType.DMA,
            pltpu.SemaphoreType.DMA,
        ],
    )
    def k(ef_h, ix_h, z_h, out_h, ix_v, b0, b1, acc, s0, s1):
        cid = lax.axis_index("c")
        sid = lax.axis_index("s")
        for i in range(nzc):
            pltpu.sync_copy(z_h, acc.at[pl.ds(sid * rows_sub + i * CH, CH)])
        rowbase = cid * (EP // 2 // CH) + sid * ROWS_W
        pltpu.sync_copy(ix_h.at[pl.ds(rowbase, ROWS_W)], ix_v)
        plsc.subcore_barrier()

        def fire(c, buf, sem):
            pltpu.async_copy(ef_h.at[pl.ds((rowbase + c) * CH, CH)], buf, sem)

        def drain(buf, sem):
            pltpu.make_async_copy(ef_h.at[pl.ds(0, CH)], buf, sem).wait()

        fire(0, b0, s0)

        def body(j, carry):
            c0 = 2 * j
            c1 = c0 + 1
            fire(c1, b1, s1)
            drain(b0, s0)
            pltpu.sync_copy(b0, acc.at[ix_v.at[c0]], add=True)

            @pl.when(j < ROWS_W // 2 - 1)
            def _():
                fire(c0 + 2, b0, s0)

            drain(b1, s1)
            pltpu.sync_copy(b1, acc.at[ix_v.at[c1]], add=True)
            return carry

        lax.fori_loop(0, ROWS_W // 2, body, 0)
        plsc.subcore_barrier()
        for i in range(nzc):
            r = sid * rows_sub + i * CH
            pltpu.sync_copy(acc.at[pl.ds(r, CH)], out_h.at[cid, pl.ds(r, CH)])

    return k(ef2, idxs2d, zeros128)


def _sc_fd_count(fracp, idxsg2d, idxdg2d, idxs2d):
    mesh = plsc.VectorSubcoreMesh(core_axis_name="c", subcore_axis_name="s")
    rows_sub = NP_ // NS

    @functools.partial(
        pl.kernel,
        out_type=(
            jax.ShapeDtypeStruct((EP, 16), jnp.float32),
            jax.ShapeDtypeStruct((NC, NP_), jnp.float32),
        ),
        mesh=mesh,
        scratch_types=[
            pltpu.VMEM((ROWS_W, CH), jnp.int32),
            pltpu.VMEM((ROWS_W, CH), jnp.int32),
            pltpu.VMEM((ROWS_W, CH), jnp.int32),
            pltpu.VMEM((CH, HIDDEN), jnp.float32),
            pltpu.VMEM((CH, HIDDEN), jnp.float32),
            pltpu.VMEM((CH, HIDDEN), jnp.float32),
            pltpu.VMEM((CH, HIDDEN), jnp.float32),
            pltpu.VMEM((CH, 16), jnp.float32),
            pltpu.VMEM((CH,), jnp.float32),
            pltpu.VMEM((rows_sub,), jnp.float32),
            pltpu.VMEM_SHARED((NP_,), jnp.float32),
            pltpu.SemaphoreType.DMA,
            pltpu.SemaphoreType.DMA,
        ],
    )
    def k(fr_h, ia_h, ib_h, is_h, fd_h, cnt_h,
          ia_v, ib_v, is_v, ba0, bb0, ba1, bb1, fdb, ones_v, zero_v, acc,
          s0, s1):
        cid = lax.axis_index("c")
        sid = lax.axis_index("s")
        wid = sid * NC + cid
        rowbase = wid * ROWS_W
        crowbase = cid * (EP // 2 // CH) + sid * ROWS_W
        for t in range(CH // 16):
            ones_v[pl.ds(t * 16, 16)] = jnp.ones((16,), jnp.float32)
        for t in range(rows_sub // 16):
            zero_v[pl.ds(t * 16, 16)] = jnp.zeros((16,), jnp.float32)
        pltpu.sync_copy(zero_v, acc.at[pl.ds(sid * rows_sub, rows_sub)])
        pltpu.sync_copy(ia_h.at[pl.ds(rowbase, ROWS_W)], ia_v)
        pltpu.sync_copy(ib_h.at[pl.ds(rowbase, ROWS_W)], ib_v)
        pltpu.sync_copy(is_h.at[pl.ds(crowbase, ROWS_W)], is_v)
        plsc.subcore_barrier()

        def fire(c, ba, bb, sem):
            pltpu.async_copy(fr_h.at[ia_v.at[c]], ba, sem)
            pltpu.async_copy(fr_h.at[ib_v.at[c]], bb, sem)

        def drain(ba, bb, sem):
            pltpu.make_async_copy(fr_h.at[pl.ds(0, CH)], ba, sem).wait()
            pltpu.make_async_copy(fr_h.at[pl.ds(0, CH)], bb, sem).wait()

        def consume(c, ba, bb):
            def rbody(r, carry):
                sv = ba[r, pl.ds(0, 16)]
                dv = bb[r, pl.ds(0, 16)]
                df = dv - sv
                fdb[r, pl.ds(0, 16)] = jnp.where(df < 0.0, df + 1.0, df)
                return carry

            lax.fori_loop(0, CH, rbody, 0)
            pltpu.sync_copy(fdb, fd_h.at[pl.ds((rowbase + c) * CH, CH)])
            pltpu.sync_copy(ones_v, acc.at[is_v.at[c]], add=True)

        fire(0, ba0, bb0, s0)

        def body(j, carry):
            c0 = 2 * j
            c1 = c0 + 1
            fire(c1, ba1, bb1, s1)
            drain(ba0, bb0, s0)
            consume(c0, ba0, bb0)

            @pl.when(j < ROWS_W // 2 - 1)
            def _():
                fire(c0 + 2, ba0, bb0, s0)

            drain(ba1, bb1, s1)
            consume(c1, ba1, bb1)
            return carry

        lax.fori_loop(0, ROWS_W // 2, body, 0)
        plsc.subcore_barrier()
        pltpu.sync_copy(
            acc.at[pl.ds(sid * rows_sub, rows_sub)],
            cnt_h.at[cid, pl.ds(sid * rows_sub, rows_sub)],
        )

    return k(fracp, idxsg2d, idxdg2d, idxs2d)


def _full(shape):
    return pl.BlockSpec(shape, lambda j: tuple(0 for _ in shape))


def _rows(bs, *rest):
    nrest = len(rest)
    return pl.BlockSpec((bs,) + rest, lambda j, _n=nrest: (j,) + (0,) * _n)


def _tc_lat(lat9p, call_t):
    def body(l_ref, c_ref, o_ref):
        L = l_ref[...]
        cols = []
        for i in range(3):
            for kk in range(3):
                acc = (
                    L[:, 3 * i + 0:3 * i + 1] * L[:, 3 * kk + 0:3 * kk + 1]
                    + L[:, 3 * i + 1:3 * i + 2] * L[:, 3 * kk + 1:3 * kk + 2]
                    + L[:, 3 * i + 2:3 * i + 3] * L[:, 3 * kk + 2:3 * kk + 3]
                )
                cols.append(acc)
        cols.append(jnp.zeros((GP, 7), jnp.float32))
        ip = jnp.concatenate(cols, axis=1)
        o_ref[...] = jnp.dot(ip, c_ref[...], preferred_element_type=jnp.float32)

    return pl.pallas_call(
        body,
        grid=(1,),
        in_specs=[_full((GP, 16)), _full((16, 4 * HIDDEN))],
        out_specs=_full((GP, 4 * HIDDEN)),
        out_shape=jax.ShapeDtypeStruct((GP, 4 * HIDDEN), jnp.float32),
    )(lat9p, call_t)


def _tc_prelude(atom, tt, n2g, glat, w):
    NB = 1024

    def body(a_ref, t_ref, g_ref, gl_ref, nw_ref, nb_ref, cw_ref, cb_ref,
             pw_ref, pb_ref, lg_ref, lb_ref, a0_ref, b0_ref, e1b_ref,
             x_ref, latn_ref, pxa_ref, pxb_ref):
        x = jnp.dot(a_ref[...], nw_ref[...], preferred_element_type=jnp.float32) + nb_ref[...]
        cond = _silu(jnp.dot(t_ref[...], cw_ref[...], preferred_element_type=jnp.float32) + cb_ref[...])
        scale = cond[:, :HIDDEN]
        shift = cond[:, HIDDEN:]
        h = jnp.dot(x, pw_ref[...], preferred_element_type=jnp.float32) + pb_ref[...]
        mu = jnp.mean(h, axis=1, keepdims=True)
        var = jnp.mean((h - mu) ** 2, axis=1, keepdims=True)
        h = (h - mu) / jnp.sqrt(var + 1e-5) * lg_ref[...] + lb_ref[...]
        h = _silu(h * scale + shift)
        x = h + x
        x_ref[...] = x
        onehot = (g_ref[...] == lax.broadcasted_iota(jnp.int32, (NB, GP), 1)).astype(jnp.float32)
        latn = jnp.dot(onehot, gl_ref[...], preferred_element_type=jnp.float32)
        latn_ref[...] = latn
        pxa_ref[...] = (
            jnp.dot(x, a0_ref[...], preferred_element_type=jnp.float32)
            + latn[:, :HIDDEN] + e1b_ref[...]
        )
        pxb_ref[...] = jnp.dot(x, b0_ref[...], preferred_element_type=jnp.float32)

    return pl.pallas_call(
        body,
        grid=(NP_ // NB,),
        in_specs=[
            _rows(NB, 128), _rows(NB, 384), _rows(NB, 1), _full((GP, 4 * HIDDEN)),
            _full((128, HIDDEN)), _full((1, HIDDEN)),
            _full((384, 2 * HIDDEN)), _full((1, 2 * HIDDEN)),
            _full((HIDDEN, HIDDEN)), _full((1, HIDDEN)),
            _full((1, HIDDEN)), _full((1, HIDDEN)),
            _full((HIDDEN, HIDDEN)), _full((HIDDEN, HIDDEN)), _full((1, HIDDEN)),
        ],
        out_specs=[
            _rows(NB, HIDDEN), _rows(NB, 4 * HIDDEN),
            _rows(NB, HIDDEN), _rows(NB, HIDDEN),
        ],
        out_shape=[
            jax.ShapeDtypeStruct((NP_, HIDDEN), jnp.float32),
            jax.ShapeDtypeStruct((NP_, 4 * HIDDEN), jnp.float32),
            jax.ShapeDtypeStruct((NP_, HIDDEN), jnp.float32),
            jax.ShapeDtypeStruct((NP_, HIDDEN), jnp.float32),
        ],
    )(atom, tt, n2g, glat, *w)


def _tc_edge(ga, gb, fd, d_t, w2_t, b2):
    EB = 512

    def body(ga_ref, gb_ref, fd_ref, d_ref, w2_ref, b2_ref, o_ref):
        theta = 2.0 * np.float32(np.pi) * fd_ref[:, :3]
        s1 = jnp.sin(theta)
        c1 = jnp.cos(theta)
        sin_cols = [[jnp.zeros((EB, 1), jnp.float32)] for _ in range(3)]
        cos_cols = [[jnp.ones((EB, 1), jnp.float32)] for _ in range(3)]
        for d in range(3):
            s1d = s1[:, d:d + 1]
            c1d = c1[:, d:d + 1]
            sk, ck = s1d, c1d
            for _k in range(1, NFREQ):
                sin_cols[d].append(sk)
                cos_cols[d].append(ck)
                sk, ck = sk * c1d + ck * s1d, ck * c1d - sk * s1d
        emb = jnp.concatenate(
            [c for d in range(3) for c in sin_cols[d]]
            + [c for d in range(3) for c in cos_cols[d]],
            axis=1,
        )
        pre = (
            ga_ref[...] + gb_ref[...]
            + jnp.dot(emb, d_ref[...], preferred_element_type=jnp.float32)
        )
        ef = _silu(pre)
        o_ref[...] = _silu(
            jnp.dot(ef, w2_ref[...], preferred_element_type=jnp.float32) + b2_ref[...]
        )

    return pl.pallas_call(
        body,
        grid=(EP // EB,),
        in_specs=[
            _rows(EB, HIDDEN), _rows(EB, HIDDEN), _rows(EB, 16),
            _full((60, HIDDEN)), _full((HIDDEN, HIDDEN)), _full((1, HIDDEN)),
        ],
        out_specs=_rows(EB, HIDDEN),
        out_shape=jax.ShapeDtypeStruct((EP, HIDDEN), jnp.float32),
    )(ga, gb, fd, d_t, w2_t, b2)


def _tc_node(parts, cnt2, x, latn, layer, w):
    NB = 1024
    n1_t, n1b, n2_t, n2b, an_t, bn_t, e1bn = w

    def body(p_ref, c_ref, x_ref, l_ref, n1_ref, n1b_ref, n2_ref, n2b_ref,
             an_ref, bn_ref, e1b_ref, xo_ref, pxa_ref, pxb_ref):
        cnt = jnp.maximum(c_ref[0, :, 0:1] + c_ref[1, :, 0:1], 1.0)
        agg = (p_ref[0] + p_ref[1]) / cnt
        x = x_ref[...]
        nin = jnp.concatenate([x, agg], axis=1)
        nf = _silu(jnp.dot(nin, n1_ref[...], preferred_element_type=jnp.float32) + n1b_ref[...])
        nf = _silu(jnp.dot(nf, n2_ref[...], preferred_element_type=jnp.float32) + n2b_ref[...])
        xn = x + nf
        xo_ref[...] = xn
        lslice = l_ref[:, (layer + 1) * HIDDEN:(layer + 2) * HIDDEN]
        pxa_ref[...] = (
            jnp.dot(xn, an_ref[...], preferred_element_type=jnp.float32)
            + lslice + e1b_ref[...]
        )
        pxb_ref[...] = jnp.dot(xn, bn_ref[...], preferred_element_type=jnp.float32)

    return pl.pallas_call(
        body,
        grid=(NP_ // NB,),
        in_specs=[
            pl.BlockSpec((2, NB, HIDDEN), lambda j: (0, j, 0)),
            pl.BlockSpec((2, NB, 8), lambda j: (0, j, 0)),
            _rows(NB, HIDDEN), _rows(NB, 4 * HIDDEN),
            _full((2 * HIDDEN, HIDDEN)), _full((1, HIDDEN)),
            _full((HIDDEN, HIDDEN)), _full((1, HIDDEN)),
            _full((HIDDEN, HIDDEN)), _full((HIDDEN, HIDDEN)), _full((1, HIDDEN)),
        ],
        out_specs=[_rows(NB, HIDDEN), _rows(NB, HIDDEN), _rows(NB, HIDDEN)],
        out_shape=[
            jax.ShapeDtypeStruct((NP_, HIDDEN), jnp.float32),
            jax.ShapeDtypeStruct((NP_, HIDDEN), jnp.float32),
            jax.ShapeDtypeStruct((NP_, HIDDEN), jnp.float32),
        ],
    )(parts, cnt2, x, latn, *w)


def _tc_node_last(parts, cnt2, x, w):
    NB = 1024
    n1_t, n1b, n2_t, n2b, type_t, type_b, coord_t = w

    def body(p_ref, c_ref, x_ref, n1_ref, n1b_ref, n2_ref, n2b_ref,
             tw_ref, tb_ref, cw_ref, xo_ref, at_ref, co_ref):
        cnt = jnp.maximum(c_ref[0, :, 0:1] + c_ref[1, :, 0:1], 1.0)
        agg = (p_ref[0] + p_ref[1]) / cnt
        x = x_ref[...]
        nin = jnp.concatenate([x, agg], axis=1)
        nf = _silu(jnp.dot(nin, n1_ref[...], preferred_element_type=jnp.float32) + n1b_ref[...])
        nf = _silu(jnp.dot(nf, n2_ref[...], preferred_element_type=jnp.float32) + n2b_ref[...])
        xn = x + nf
        xo_ref[...] = xn
        at_ref[...] = jnp.dot(xn, tw_ref[...], preferred_element_type=jnp.float32) + tb_ref[...]
        co_ref[...] = jnp.dot(xn, cw_ref[...], preferred_element_type=jnp.float32)

    return pl.pallas_call(
        body,
        grid=(NP_ // NB,),
        in_specs=[
            pl.BlockSpec((2, NB, HIDDEN), lambda j: (0, j, 0)),
            pl.BlockSpec((2, NB, 8), lambda j: (0, j, 0)),
            _rows(NB, HIDDEN),
            _full((2 * HIDDEN, HIDDEN)), _full((1, HIDDEN)),
            _full((HIDDEN, HIDDEN)), _full((1, HIDDEN)),
            _full((HIDDEN, 104)), _full((1, 104)), _full((HIDDEN, 8)),
        ],
        out_specs=[_rows(NB, HIDDEN), _rows(NB, 104), _rows(NB, 8)],
        out_shape=[
            jax.ShapeDtypeStruct((NP_, HIDDEN), jnp.float32),
            jax.ShapeDtypeStruct((NP_, 104), jnp.float32),
            jax.ShapeDtypeStruct((NP_, 8), jnp.float32),
        ],
    )(parts, cnt2, x, *w)


def _tc_gpool(x, n2g):
    NB = 1024

    def body(x_ref, g_ref, o_ref):
        j = pl.program_id(0)

        @pl.when(j == 0)
        def _():
            o_ref[...] = jnp.zeros_like(o_ref)

        onehot = (g_ref[...] == lax.broadcasted_iota(jnp.int32, (NB, GP), 1)).astype(jnp.float32)
        gf = lax.dot_general(
            onehot, x_ref[...],
            dimension_numbers=(((0,), (0,)), ((), ())),
            preferred_element_type=jnp.float32,
        )
        gc = jnp.sum(onehot, axis=0)[:, None]
        o_ref[...] += jnp.concatenate(
            [gf, jnp.broadcast_to(gc, (GP, HIDDEN))], axis=1
        )

    return pl.pallas_call(
        body,
        grid=(NP_ // NB,),
        in_specs=[_rows(NB, HIDDEN), _rows(NB, 1)],
        out_specs=_full((GP, 2 * HIDDEN)),
        out_shape=jax.ShapeDtypeStruct((GP, 2 * HIDDEN), jnp.float32),
    )(x, n2g)


def _tc_latt_head(gfc, lat9p, lattw_t):
    def body(g_ref, l_ref, w_ref, o_ref):
        gfc = g_ref[...]
        gfeat = gfc[:, :HIDDEN] / jnp.maximum(gfc[:, HIDDEN:HIDDEN + 1], 1.0)
        t9 = jnp.dot(gfeat, w_ref[...], preferred_element_type=jnp.float32)
        L = l_ref[...]
        cols = []
        for i in range(3):
            for kk in range(3):
                acc = (
                    t9[:, 3 * i + 0:3 * i + 1] * L[:, 0 + kk:1 + kk]
                    + t9[:, 3 * i + 1:3 * i + 2] * L[:, 3 + kk:4 + kk]
                    + t9[:, 3 * i + 2:3 * i + 3] * L[:, 6 + kk:7 + kk]
                )
                cols.append(acc)
        cols.append(jnp.zeros((GP, 7), jnp.float32))
        o_ref[...] = jnp.concatenate(cols, axis=1)

    return pl.pallas_call(
        body,
        grid=(1,),
        in_specs=[_full((GP, 2 * HIDDEN)), _full((GP, 16)), _full((HIDDEN, 16))],
        out_specs=_full((GP, 16)),
        out_shape=jax.ShapeDtypeStruct((GP, 16), jnp.float32),
    )(gfc, lat9p, lattw_t)


def kernel(atom_types, frac_coords, lattices, time_embeds, text_embeds,
           params, edge_index, edge2graph, node2graph):
    p = params
    f32 = jnp.float32

    src = edge_index[0]
    dst = edge_index[1]
    pad_e = EP - E
    srcg = jnp.pad(src, (0, pad_e)).reshape(EP // CH, CH)
    dstg = jnp.pad(dst, (0, pad_e)).reshape(EP // CH, CH)
    srcs = jnp.pad(src, (0, pad_e), constant_values=N).reshape(EP // CH, CH)

    pad_n = NP_ - N
    atom = jnp.pad(atom_types, ((0, pad_n), (0, 128 - atom_types.shape[1])))
    tt = jnp.pad(
        jnp.concatenate([time_embeds, text_embeds], axis=1), ((0, pad_n), (0, 0))
    )
    n2g = jnp.pad(node2graph, (0, pad_n), constant_values=GP - 1)[:, None]
    fracp = jnp.pad(frac_coords, ((0, pad_n), (0, 128 - 3)))
    lat9p = jnp.pad(lattices.reshape(NGRAPH, 9), ((0, GP - NGRAPH), (0, 7)))
    zeros128 = jnp.zeros((CH, HIDDEN), f32)

    A_t, B_t, D_t, e1b = [], [], [], []
    call_rows = []
    for i in range(NLAYERS):
        W1 = p[f'e1_W_{i}']
        A_t.append(W1[:, :HIDDEN].T)
        B_t.append(W1[:, HIDDEN:2 * HIDDEN].T)
        call_rows.append(W1[:, 2 * HIDDEN:2 * HIDDEN + 9])
        D_t.append(W1[:, 2 * HIDDEN + 9:].T)
        e1b.append(p[f'e1_b_{i}'][None, :])
    call_t = jnp.pad(
        jnp.concatenate(call_rows, axis=0).T, ((0, 7), (0, 0))
    )

    prelude_w = [
        jnp.pad(p['node_W'].T, ((0, 128 - 103), (0, 0))), p['node_b'][None, :],
        p['cond_W'].T, p['cond_b'][None, :],
        p['proj_W'].T, p['proj_b'][None, :],
        p['ln_g'][None, :], p['ln_b'][None, :],
        A_t[0], B_t[0], e1b[0],
    ]
    type_t = jnp.pad(p['type_W'].T, ((0, 0), (0, 1)))
    type_b = jnp.pad(p['type_b'], (0, 1))[None, :]
    coord_t = jnp.pad(p['coord_W'].T, ((0, 0), (0, 5)))
    lattw_t = jnp.pad(p['latt_W'].T, ((0, 0), (0, 7)))

    glat = _tc_lat(lat9p, call_t)
    x, latn, pxa, pxb = _tc_prelude(atom, tt, n2g, glat, prelude_w)
    fd, cnt2 = _sc_fd_count(fracp, srcg, dstg, srcs)
    cnt2 = jnp.broadcast_to(cnt2[:, :, None], (NC, NP_, 8))

    for i in range(NLAYERS):
        ga, gb = _sc_gather2(pxa, pxb, srcg, dstg, HIDDEN)
        ef2 = _tc_edge(ga, gb, fd, D_t[i], p[f'e2_W_{i}'].T,
                       p[f'e2_b_{i}'][None, :])
        parts = _sc_scatter(ef2, srcs, zeros128)
        if i < NLAYERS - 1:
            nw = [p[f'n1_W_{i}'].T, p[f'n1_b_{i}'][None, :],
                  p[f'n2_W_{i}'].T, p[f'n2_b_{i}'][None, :],
                  A_t[i + 1], B_t[i + 1], e1b[i + 1]]
            x, pxa, pxb = _tc_node(parts, cnt2, x, latn, i, nw)
        else:
            nw = [p[f'n1_W_{i}'].T, p[f'n1_b_{i}'][None, :],
                  p[f'n2_W_{i}'].T, p[f'n2_b_{i}'][None, :],
                  type_t, type_b, coord_t]
            x, atom_out, coords = _tc_node_last(parts, cnt2, x, nw)

    gfc = _tc_gpool(x, n2g)
    latt9 = _tc_latt_head(gfc, lat9p, lattw_t)

    atom_types_out = atom_out[:N, :103]
    coords_out = coords[:N, :3]
    x_out = x[:N]
    latt = latt9[:NGRAPH, :9].reshape(NGRAPH, 3, 3)
    return (atom_types_out, latt, coords_out, x_out)


# speedup vs baseline: 3.2942x; 3.2942x over previous
import functools
import numpy as np
import jax
import jax.numpy as jnp
from jax import lax
from jax.experimental import pallas as pl
from jax.experimental.pallas import tpu as pltpu
from jax.experimental.pallas import tpu_sc as plsc

N = 10000
E = 320000
NGRAPH = 500
HIDDEN = 128
NFREQ = 10
NLAYERS = 4

NP_ = 10240
EP = 327680
GP = 512
NC = 2
NS = 16
NW = NC * NS
CH = 128
UN = 4
PERW = EP // NW
ROWS_W = PERW // CH
NSUP = ROWS_W // UN


def _silu(v):
    return v * jax.nn.sigmoid(v)


def _sc_gather2(table_a, table_b, idxa2d, idxb2d, width):
    mesh = plsc.VectorSubcoreMesh(core_axis_name="c", subcore_axis_name="s")

    @functools.partial(
        pl.kernel,
        out_type=(
            jax.ShapeDtypeStruct((EP, width), jnp.float32),
            jax.ShapeDtypeStruct((EP, width), jnp.float32),
        ),
        mesh=mesh,
        scratch_types=[
            pltpu.VMEM((ROWS_W, CH), jnp.int32),
            pltpu.VMEM((ROWS_W, CH), jnp.int32),
            pltpu.VMEM((CH, width), jnp.float32),
            pltpu.VMEM((CH, width), jnp.float32),
            pltpu.VMEM((CH, width), jnp.float32),
            pltpu.VMEM((CH, width), jnp.float32),
            pltpu.SemaphoreType.DMA,
            pltpu.SemaphoreType.DMA,
        ],
    )
    def k(ta_h, tb_h, ia_h, ib_h, oa_h, ob_h, ia_v, ib_v,
          ba0, bb0, ba1, bb1, s0, s1):
        cid = lax.axis_index("c")
        sid = lax.axis_index("s")
        wid = sid * NC + cid
        rowbase = wid * ROWS_W
        pltpu.sync_copy(ia_h.at[pl.ds(rowbase, ROWS_W)], ia_v)
        pltpu.sync_copy(ib_h.at[pl.ds(rowbase, ROWS_W)], ib_v)

        def fire(c, ba, bb, sem):
            pltpu.async_copy(ta_h.at[ia_v.at[c]], ba, sem)
            pltpu.async_copy(tb_h.at[ib_v.at[c]], bb, sem)

        def drain(ba, bb, sem):
            pltpu.make_async_copy(ta_h.at[pl.ds(0, CH)], ba, sem).wait()
            pltpu.make_async_copy(ta_h.at[pl.ds(0, CH)], bb, sem).wait()

        fire(0, ba0, bb0, s0)

        def body(j, carry):
            c0 = 2 * j
            c1 = c0 + 1
            fire(c1, ba1, bb1, s1)
            drain(ba0, bb0, s0)
            base0 = (rowbase + c0) * CH
            pltpu.sync_copy(ba0, oa_h.at[pl.ds(base0, CH)])
            pltpu.sync_copy(bb0, ob_h.at[pl.ds(base0, CH)])

            @pl.when(j < ROWS_W // 2 - 1)
            def _():
                fire(c0 + 2, ba0, bb0, s0)

            drain(ba1, bb1, s1)
            base1 = (rowbase + c1) * CH
            pltpu.sync_copy(ba1, oa_h.at[pl.ds(base1, CH)])
            pltpu.sync_copy(bb1, ob_h.at[pl.ds(base1, CH)])
            return carry

        lax.fori_loop(0, ROWS_W // 2, body, 0)

    return k(table_a, table_b, idxa2d, idxb2d)


def _sc_scatter(ef2, idxs2d, zeros128):
    mesh = plsc.VectorSubcoreMesh(core_axis_name="c", subcore_axis_name="s")
    rows_sub = NP_ // NS
    nzc = rows_sub // CH

    @functools.partial(
        pl.kernel,
        out_type=jax.ShapeDtypeStruct((NC, NP_, HIDDEN), jnp.float32),
        mesh=mesh,
        scratch_types=[
            pltpu.VMEM((ROWS_W, CH), jnp.int32),
            pltpu.VMEM((CH, HIDDEN), jnp.float32),
            pltpu.VMEM((CH, HIDDEN), jnp.float32),
            pltpu.VMEM_SHARED((NP_, HIDDEN), jnp.float32),
            pltpu.SemaphoreType.DMA,
            pltpu.SemaphoreType.DMA,
        ],
    )
    def k(ef_h, ix_h, z_h, out_h, ix_v, b0, b1, acc, s0, s1):
        cid = lax.axis_index("c")
        sid = lax.axis_index("s")
        for i in range(nzc):
            pltpu.sync_copy(z_h, acc.at[pl.ds(sid * rows_sub + i * CH, CH)])
        rowbase = cid * (EP // 2 // CH) + sid * ROWS_W
        pltpu.sync_copy(ix_h.at[pl.ds(rowbase, ROWS_W)], ix_v)
        plsc.subcore_barrier()

        def fire(c, buf, sem):
            pltpu.async_copy(ef_h.at[pl.ds((rowbase + c) * CH, CH)], buf, sem)

        def drain(buf, sem):
            pltpu.make_async_copy(ef_h.at[pl.ds(0, CH)], buf, sem).wait()

        fire(0, b0, s0)

        def body(j, carry):
            c0 = 2 * j
            c1 = c0 + 1
            fire(c1, b1, s1)
            drain(b0, s0)
            pltpu.sync_copy(b0, acc.at[ix_v.at[c0]], add=True)

            @pl.when(j < ROWS_W // 2 - 1)
            def _():
                fire(c0 + 2, b0, s0)

            drain(b1, s1)
            pltpu.sync_copy(b1, acc.at[ix_v.at[c1]], add=True)
            return carry

        lax.fori_loop(0, ROWS_W // 2, body, 0)
        plsc.subcore_barrier()
        for i in range(nzc):
            r = sid * rows_sub + i * CH
            pltpu.sync_copy(acc.at[pl.ds(r, CH)], out_h.at[cid, pl.ds(r, CH)])

    return k(ef2, idxs2d, zeros128)


def _sc_fd_count(fracp, idxsg2d, idxdg2d, idxs2d):
    mesh = plsc.VectorSubcoreMesh(core_axis_name="c", subcore_axis_name="s")
    rows_sub = NP_ // NS

    @functools.partial(
        pl.kernel,
        out_type=(
            jax.ShapeDtypeStruct((EP, 16), jnp.float32),
            jax.ShapeDtypeStruct((NC, NP_), jnp.float32),
        ),
        mesh=mesh,
        scratch_types=[
            pltpu.VMEM((ROWS_W, CH), jnp.int32),
            pltpu.VMEM((ROWS_W, CH), jnp.int32),
            pltpu.VMEM((ROWS_W, CH), jnp.int32),
            pltpu.VMEM((CH, HIDDEN), jnp.float32),
            pltpu.VMEM((CH, HIDDEN), jnp.float32),
            pltpu.VMEM((CH, HIDDEN), jnp.float32),
            pltpu.VMEM((CH, HIDDEN), jnp.float32),
            pltpu.VMEM((CH, 16), jnp.float32),
            pltpu.VMEM((CH,), jnp.float32),
            pltpu.VMEM((rows_sub,), jnp.float32),
            pltpu.VMEM_SHARED((NP_,), jnp.float32),
            pltpu.SemaphoreType.DMA,
            pltpu.SemaphoreType.DMA,
        ],
    )
    def k(fr_h, ia_h, ib_h, is_h, fd_h, cnt_h,
          ia_v, ib_v, is_v, ba0, bb0, ba1, bb1, fdb, ones_v, zero_v, acc,
          s0, s1):
        cid = lax.axis_index("c")
        sid = lax.axis_index("s")
        wid = sid * NC + cid
        rowbase = wid * ROWS_W
        crowbase = cid * (EP // 2 // CH) + sid * ROWS_W
        for t in range(CH // 16):
            ones_v[pl.ds(t * 16, 16)] = jnp.ones((16,), jnp.float32)
        for t in range(rows_sub // 16):
            zero_v[pl.ds(t * 16, 16)] = jnp.zeros((16,), jnp.float32)
        pltpu.sync_copy(zero_v, acc.at[pl.ds(sid * rows_sub, rows_sub)])
        pltpu.sync_copy(ia_h.at[pl.ds(rowbase, ROWS_W)], ia_v)
        pltpu.sync_copy(ib_h.at[pl.ds(rowbase, ROWS_W)], ib_v)
        pltpu.sync_copy(is_h.at[pl.ds(crowbase, ROWS_W)], is_v)
        plsc.subcore_barrier()

        def fire(c, ba, bb, sem):
            pltpu.async_copy(fr_h.at[ia_v.at[c]], ba, sem)
            pltpu.async_copy(fr_h.at[ib_v.at[c]], bb, sem)

        def drain(ba, bb, sem):
            pltpu.make_async_copy(fr_h.at[pl.ds(0, CH)], ba, sem).wait()
            pltpu.make_async_copy(fr_h.at[pl.ds(0, CH)], bb, sem).wait()

        def consume(c, ba, bb):
            def rbody(r, carry):
                sv = ba[r, pl.ds(0, 16)]
                dv = bb[r, pl.ds(0, 16)]
                df = dv - sv
                fdb[r, pl.ds(0, 16)] = jnp.where(df < 0.0, df + 1.0, df)
                return carry

            lax.fori_loop(0, CH, rbody, 0)
            pltpu.sync_copy(fdb, fd_h.at[pl.ds((rowbase + c) * CH, CH)])
            pltpu.sync_copy(ones_v, acc.at[is_v.at[c]], add=True)

        fire(0, ba0, bb0, s0)

        def body(j, carry):
            c0 = 2 * j
            c1 = c0 + 1
            fire(c1, ba1, bb1, s1)
            drain(ba0, bb0, s0)
            consume(c0, ba0, bb0)

            @pl.when(j < ROWS_W // 2 - 1)
            def _():
                fire(c0 + 2, ba0, bb0, s0)

            drain(ba1, bb1, s1)
            consume(c1, ba1, bb1)
            return carry

        lax.fori_loop(0, ROWS_W // 2, body, 0)
        plsc.subcore_barrier()
        pltpu.sync_copy(
            acc.at[pl.ds(sid * rows_sub, rows_sub)],
            cnt_h.at[cid, pl.ds(sid * rows_sub, rows_sub)],
        )

    return k(fracp, idxsg2d, idxdg2d, idxs2d)


def _full(shape):
    return pl.BlockSpec(shape, lambda j: tuple(0 for _ in shape))


def _rows(bs, *rest):
    nrest = len(rest)
    return pl.BlockSpec((bs,) + rest, lambda j, _n=nrest: (j,) + (0,) * _n)


def _tc_lat(lat9p, call_t):
    def body(l_ref, c_ref, o_ref):
        L = l_ref[...]
        cols = []
        for i in range(3):
            for kk in range(3):
                acc = (
                    L[:, 3 * i + 0:3 * i + 1] * L[:, 3 * kk + 0:3 * kk + 1]
                    + L[:, 3 * i + 1:3 * i + 2] * L[:, 3 * kk + 1:3 * kk + 2]
                    + L[:, 3 * i + 2:3 * i + 3] * L[:, 3 * kk + 2:3 * kk + 3]
                )
                cols.append(acc)
        cols.append(jnp.zeros((GP, 7), jnp.float32))
        ip = jnp.concatenate(cols, axis=1)
        o_ref[...] = jnp.dot(ip, c_ref[...], preferred_element_type=jnp.float32)

    return pl.pallas_call(
        body,
        grid=(1,),
        in_specs=[_full((GP, 16)), _full((16, 4 * HIDDEN))],
        out_specs=_full((GP, 4 * HIDDEN)),
        out_shape=jax.ShapeDtypeStruct((GP, 4 * HIDDEN), jnp.float32),
    )(lat9p, call_t)


def _tc_prelude(atom, tt, n2g, glat, w):
    NB = 1024

    def body(a_ref, t_ref, g_ref, gl_ref, nw_ref, nb_ref, cw_ref, cb_ref,
             pw_ref, pb_ref, lg_ref, lb_ref, a0_ref, b0_ref, e1b_ref,
             x_ref, latn_ref, pxa_ref, pxb_ref):
        x = jnp.dot(a_ref[...], nw_ref[...], preferred_element_type=jnp.float32) + nb_ref[...]
        cond = _silu(jnp.dot(t_ref[...], cw_ref[...], preferred_element_type=jnp.float32) + cb_ref[...])
        scale = cond[:, :HIDDEN]
        shift = cond[:, HIDDEN:]
        h = jnp.dot(x, pw_ref[...], preferred_element_type=jnp.float32) + pb_ref[...]
        mu = jnp.mean(h, axis=1, keepdims=True)
        var = jnp.mean((h - mu) ** 2, axis=1, keepdims=True)
        h = (h - mu) / jnp.sqrt(var + 1e-5) * lg_ref[...] + lb_ref[...]
        h = _silu(h * scale + shift)
        x = h + x
        x_ref[...] = x
        onehot = (g_ref[...] == lax.broadcasted_iota(jnp.int32, (NB, GP), 1)).astype(jnp.float32)
        latn = jnp.dot(onehot, gl_ref[...], preferred_element_type=jnp.float32)
        latn_ref[...] = latn
        pxa_ref[...] = (
            jnp.dot(x, a0_ref[...], preferred_element_type=jnp.float32)
            + latn[:, :HIDDEN] + e1b_ref[...]
        )
        pxb_ref[...] = jnp.dot(x, b0_ref[...], preferred_element_type=jnp.float32)

    return pl.pallas_call(
        body,
        grid=(NP_ // NB,),
        in_specs=[
            _rows(NB, 128), _rows(NB, 384), _rows(NB, 1), _full((GP, 4 * HIDDEN)),
            _full((128, HIDDEN)), _full((1, HIDDEN)),
            _full((384, 2 * HIDDEN)), _full((1, 2 * HIDDEN)),
            _full((HIDDEN, HIDDEN)), _full((1, HIDDEN)),
            _full((1, HIDDEN)), _full((1, HIDDEN)),
            _full((HIDDEN, HIDDEN)), _full((HIDDEN, HIDDEN)), _full((1, HIDDEN)),
        ],
        out_specs=[
            _rows(NB, HIDDEN), _rows(NB, 4 * HIDDEN),
            _rows(NB, HIDDEN), _rows(NB, HIDDEN),
        ],
        out_shape=[
            jax.ShapeDtypeStruct((NP_, HIDDEN), jnp.float32),
            jax.ShapeDtypeStruct((NP_, 4 * HIDDEN), jnp.float32),
            jax.ShapeDtypeStruct((NP_, HIDDEN), jnp.float32),
            jax.ShapeDtypeStruct((NP_, HIDDEN), jnp.float32),
        ],
    )(atom, tt, n2g, glat, *w)


def _tc_emb(fd):
    EB = 2048

    def body(fd_ref, o_ref):
        freqs_row = 2.0 * np.float32(np.pi) * lax.broadcasted_iota(
            jnp.int32, (1, NFREQ), 1).astype(jnp.float32)
        df = fd_ref[...]
        m = jnp.concatenate(
            [df[:, d:d + 1] * freqs_row for d in range(3)], axis=1
        )
        o_ref[...] = jnp.concatenate(
            [jnp.sin(m), jnp.cos(m), jnp.zeros((EB, 4), jnp.float32)], axis=1
        )

    return pl.pallas_call(
        body,
        grid=(EP // EB,),
        in_specs=[_rows(EB, 16)],
        out_specs=_rows(EB, 64),
        out_shape=jax.ShapeDtypeStruct((EP, 64), jnp.float32),
    )(fd)


def _tc_edge(ga, gb, emb, d_t, w2_t, b2):
    EB = 2048

    def body(ga_ref, gb_ref, e_ref, d_ref, w2_ref, b2_ref, o_ref):
        pre = (
            ga_ref[...] + gb_ref[...]
            + jnp.dot(e_ref[...], d_ref[...], preferred_element_type=jnp.float32)
        )
        ef = _silu(pre)
        o_ref[...] = _silu(
            jnp.dot(ef, w2_ref[...], preferred_element_type=jnp.float32) + b2_ref[...]
        )

    return pl.pallas_call(
        body,
        grid=(EP // EB,),
        in_specs=[
            _rows(EB, HIDDEN), _rows(EB, HIDDEN), _rows(EB, 64),
            _full((64, HIDDEN)), _full((HIDDEN, HIDDEN)), _full((1, HIDDEN)),
        ],
        out_specs=_rows(EB, HIDDEN),
        out_shape=jax.ShapeDtypeStruct((EP, HIDDEN), jnp.float32),
    )(ga, gb, emb, d_t, w2_t, b2)


def _tc_node(parts, cnt2, x, latn, layer, w):
    NB = 1024
    n1_t, n1b, n2_t, n2b, an_t, bn_t, e1bn = w

    def body(p_ref, c_ref, x_ref, l_ref, n1_ref, n1b_ref, n2_ref, n2b_ref,
             an_ref, bn_ref, e1b_ref, xo_ref, pxa_ref, pxb_ref):
        cnt = jnp.maximum(c_ref[0, :, 0:1] + c_ref[1, :, 0:1], 1.0)
        agg = (p_ref[0] + p_ref[1]) / cnt
        x = x_ref[...]
        nin = jnp.concatenate([x, agg], axis=1)
        nf = _silu(jnp.dot(nin, n1_ref[...], preferred_element_type=jnp.float32) + n1b_ref[...])
        nf = _silu(jnp.dot(nf, n2_ref[...], preferred_element_type=jnp.float32) + n2b_ref[...])
        xn = x + nf
        xo_ref[...] = xn
        lslice = l_ref[:, (layer + 1) * HIDDEN:(layer + 2) * HIDDEN]
        pxa_ref[...] = (
            jnp.dot(xn, an_ref[...], preferred_element_type=jnp.float32)
            + lslice + e1b_ref[...]
        )
        pxb_ref[...] = jnp.dot(xn, bn_ref[...], preferred_element_type=jnp.float32)

    return pl.pallas_call(
        body,
        grid=(NP_ // NB,),
        in_specs=[
            pl.BlockSpec((2, NB, HIDDEN), lambda j: (0, j, 0)),
            pl.BlockSpec((2, NB, 8), lambda j: (0, j, 0)),
            _rows(NB, HIDDEN), _rows(NB, 4 * HIDDEN),
            _full((2 * HIDDEN, HIDDEN)), _full((1, HIDDEN)),
            _full((HIDDEN, HIDDEN)), _full((1, HIDDEN)),
            _full((HIDDEN, HIDDEN)), _full((HIDDEN, HIDDEN)), _full((1, HIDDEN)),
        ],
        out_specs=[_rows(NB, HIDDEN), _rows(NB, HIDDEN), _rows(NB, HIDDEN)],
        out_shape=[
            jax.ShapeDtypeStruct((NP_, HIDDEN), jnp.float32),
            jax.ShapeDtypeStruct((NP_, HIDDEN), jnp.float32),
            jax.ShapeDtypeStruct((NP_, HIDDEN), jnp.float32),
        ],
    )(parts, cnt2, x, latn, *w)


def _tc_node_last(parts, cnt2, x, w):
    NB = 1024
    n1_t, n1b, n2_t, n2b, type_t, type_b, coord_t = w

    def body(p_ref, c_ref, x_ref, n1_ref, n1b_ref, n2_ref, n2b_ref,
             tw_ref, tb_ref, cw_ref, xo_ref, at_ref, co_ref):
        cnt = jnp.maximum(c_ref[0, :, 0:1] + c_ref[1, :, 0:1], 1.0)
        agg = (p_ref[0] + p_ref[1]) / cnt
        x = x_ref[...]
        nin = jnp.concatenate([x, agg], axis=1)
        nf = _silu(jnp.dot(nin, n1_ref[...], preferred_element_type=jnp.float32) + n1b_ref[...])
        nf = _silu(jnp.dot(nf, n2_ref[...], preferred_element_type=jnp.float32) + n2b_ref[...])
        xn = x + nf
        xo_ref[...] = xn
        at_ref[...] = jnp.dot(xn, tw_ref[...], preferred_element_type=jnp.float32) + tb_ref[...]
        co_ref[...] = jnp.dot(xn, cw_ref[...], preferred_element_type=jnp.float32)

    return pl.pallas_call(
        body,
        grid=(NP_ // NB,),
        in_specs=[
            pl.BlockSpec((2, NB, HIDDEN), lambda j: (0, j, 0)),
            pl.BlockSpec((2, NB, 8), lambda j: (0, j, 0)),
            _rows(NB, HIDDEN),
            _full((2 * HIDDEN, HIDDEN)), _full((1, HIDDEN)),
            _full((HIDDEN, HIDDEN)), _full((1, HIDDEN)),
            _full((HIDDEN, 104)), _full((1, 104)), _full((HIDDEN, 8)),
        ],
        out_specs=[_rows(NB, HIDDEN), _rows(NB, 104), _rows(NB, 8)],
        out_shape=[
            jax.ShapeDtypeStruct((NP_, HIDDEN), jnp.float32),
            jax.ShapeDtypeStruct((NP_, 104), jnp.float32),
            jax.ShapeDtypeStruct((NP_, 8), jnp.float32),
        ],
    )(parts, cnt2, x, *w)


def _tc_gpool(x, n2g):
    NB = 1024

    def body(x_ref, g_ref, o_ref):
        j = pl.program_id(0)

        @pl.when(j == 0)
        def _():
            o_ref[...] = jnp.zeros_like(o_ref)

        onehot = (g_ref[...] == lax.broadcasted_iota(jnp.int32, (NB, GP), 1)).astype(jnp.float32)
        gf = lax.dot_general(
            onehot, x_ref[...],
            dimension_numbers=(((0,), (0,)), ((), ())),
            preferred_element_type=jnp.float32,
        )
        gc = jnp.sum(onehot, axis=0)[:, None]
        o_ref[...] += jnp.concatenate(
            [gf, jnp.broadcast_to(gc, (GP, HIDDEN))], axis=1
        )

    return pl.pallas_call(
        body,
        grid=(NP_ // NB,),
        in_specs=[_rows(NB, HIDDEN), _rows(NB, 1)],
        out_specs=_full((GP, 2 * HIDDEN)),
        out_shape=jax.ShapeDtypeStruct((GP, 2 * HIDDEN), jnp.float32),
    )(x, n2g)


def _tc_latt_head(gfc, lat9p, lattw_t):
    def body(g_ref, l_ref, w_ref, o_ref):
        gfc = g_ref[...]
        gfeat = gfc[:, :HIDDEN] / jnp.maximum(gfc[:, HIDDEN:HIDDEN + 1], 1.0)
        t9 = jnp.dot(gfeat, w_ref[...], preferred_element_type=jnp.float32)
        L = l_ref[...]
        cols = []
        for i in range(3):
            for kk in range(3):
                acc = (
                    t9[:, 3 * i + 0:3 * i + 1] * L[:, 0 + kk:1 + kk]
                    + t9[:, 3 * i + 1:3 * i + 2] * L[:, 3 + kk:4 + kk]
                    + t9[:, 3 * i + 2:3 * i + 3] * L[:, 6 + kk:7 + kk]
                )
                cols.append(acc)
        cols.append(jnp.zeros((GP, 7), jnp.float32))
        o_ref[...] = jnp.concatenate(cols, axis=1)

    return pl.pallas_call(
        body,
        grid=(1,),
        in_specs=[_full((GP, 2 * HIDDEN)), _full((GP, 16)), _full((HIDDEN, 16))],
        out_specs=_full((GP, 16)),
        out_shape=jax.ShapeDtypeStruct((GP, 16), jnp.float32),
    )(gfc, lat9p, lattw_t)


def kernel(atom_types, frac_coords, lattices, time_embeds, text_embeds,
           params, edge_index, edge2graph, node2graph):
    p = params
    f32 = jnp.float32

    src = edge_index[0]
    dst = edge_index[1]
    pad_e = EP - E
    srcg = jnp.pad(src, (0, pad_e)).reshape(EP // CH, CH)
    dstg = jnp.pad(dst, (0, pad_e)).reshape(EP // CH, CH)
    srcs = jnp.pad(src, (0, pad_e), constant_values=N).reshape(EP // CH, CH)

    pad_n = NP_ - N
    atom = jnp.pad(atom_types, ((0, pad_n), (0, 128 - atom_types.shape[1])))
    tt = jnp.pad(
        jnp.concatenate([time_embeds, text_embeds], axis=1), ((0, pad_n), (0, 0))
    )
    n2g = jnp.pad(node2graph, (0, pad_n), constant_values=GP - 1)[:, None]
    fracp = jnp.pad(frac_coords, ((0, pad_n), (0, 128 - 3)))
    lat9p = jnp.pad(lattices.reshape(NGRAPH, 9), ((0, GP - NGRAPH), (0, 7)))
    zeros128 = jnp.zeros((CH, HIDDEN), f32)

    A_t, B_t, D_t, e1b = [], [], [], []
    call_rows = []
    for i in range(NLAYERS):
        W1 = p[f'e1_W_{i}']
        A_t.append(W1[:, :HIDDEN].T)
        B_t.append(W1[:, HIDDEN:2 * HIDDEN].T)
        call_rows.append(W1[:, 2 * HIDDEN:2 * HIDDEN + 9])
        D_t.append(W1[:, 2 * HIDDEN + 9:].T)
        e1b.append(p[f'e1_b_{i}'][None, :])
    call_t = jnp.pad(
        jnp.concatenate(call_rows, axis=0).T, ((0, 7), (0, 0))
    )
    d64_t = [jnp.pad(d, ((0, 4), (0, 0))) for d in D_t]

    prelude_w = [
        jnp.pad(p['node_W'].T, ((0, 128 - 103), (0, 0))), p['node_b'][None, :],
        p['cond_W'].T, p['cond_b'][None, :],
        p['proj_W'].T, p['proj_b'][None, :],
        p['ln_g'][None, :], p['ln_b'][None, :],
        A_t[0], B_t[0], e1b[0],
    ]
    type_t = jnp.pad(p['type_W'].T, ((0, 0), (0, 1)))
    type_b = jnp.pad(p['type_b'], (0, 1))[None, :]
    coord_t = jnp.pad(p['coord_W'].T, ((0, 0), (0, 5)))
    lattw_t = jnp.pad(p['latt_W'].T, ((0, 0), (0, 7)))

    glat = _tc_lat(lat9p, call_t)
    x, latn, pxa, pxb = _tc_prelude(atom, tt, n2g, glat, prelude_w)
    fd, cnt2 = _sc_fd_count(fracp, srcg, dstg, srcs)
    emb = _tc_emb(fd)
    cnt2 = jnp.broadcast_to(cnt2[:, :, None], (NC, NP_, 8))

    for i in range(NLAYERS):
        ga, gb = _sc_gather2(pxa, pxb, srcg, dstg, HIDDEN)
        ef2 = _tc_edge(ga, gb, emb, d64_t[i], p[f'e2_W_{i}'].T,
                       p[f'e2_b_{i}'][None, :])
        parts = _sc_scatter(ef2, srcs, zeros128)
        if i < NLAYERS - 1:
            nw = [p[f'n1_W_{i}'].T, p[f'n1_b_{i}'][None, :],
                  p[f'n2_W_{i}'].T, p[f'n2_b_{i}'][None, :],
                  A_t[i + 1], B_t[i + 1], e1b[i + 1]]
            x, pxa, pxb = _tc_node(parts, cnt2, x, latn, i, nw)
        else:
            nw = [p[f'n1_W_{i}'].T, p[f'n1_b_{i}'][None, :],
                  p[f'n2_W_{i}'].T, p[f'n2_b_{i}'][None, :],
                  type_t, type_b, coord_t]
            x, atom_out, coords = _tc_node_last(parts, cnt2, x, nw)

    gfc = _tc_gpool(x, n2g)
    latt9 = _tc_latt_head(gfc, lat9p, lattw_t)

    atom_types_out = atom_out[:N, :103]
    coords_out = coords[:N, :3]
    x_out = x[:N]
    latt = latt9[:NGRAPH, :9].reshape(NGRAPH, 3, 3)
    return (atom_types_out, latt, coords_out, x_out)
